# baseline (speedup 1.0000x reference)
import functools

import jax
import jax.numpy as jnp
from jax import lax
from jax.experimental import pallas as pl
from jax.experimental.pallas import tpu as pltpu
from jax.experimental.pallas import tpu_sc as plsc

_N = 10000
_E = 320000
_D = 128
_NC = 2
_NS = 16
_NW = _NC * _NS
_EPT = _E // _NW
_CH = 80
_NCHUNK = 126
_EPAD = _NCHUNK * _CH
_ZB = 80
_STRIPE = 640
_LAST_COPIES = (_N - (_NS - 1) * _STRIPE) // _ZB
_L = 16


def _gate_body(x_ref, w_ref, b_ref, g_ref):
    g_ref[...] = lax.dot_general(
        x_ref[...], w_ref[...], (((1,), (1,)), ((), ())),
        preferred_element_type=jnp.float32) + b_ref[...]


def _sum_body(p_ref, o_ref):
    o_ref[...] = p_ref[0] + p_ref[1]


def _sc_body(x_hbm, g1_hbm, g2_hbm, pk_hbm, out_hbm,
             pk_a, pk_b, g1_v, g2_v, val_v, buf_a, buf_b, acc,
             sem_a, sem_b, ssem_a, ssem_b):
    c = lax.axis_index("c")
    s = lax.axis_index("s")
    wid = c * _NS + s

    pltpu.sync_copy(g1_hbm, g1_v)
    pltpu.sync_copy(g2_hbm, g2_v)

    zf = jnp.zeros((_L,), jnp.float32)

    @pl.loop(0, _ZB)
    def _zero_buf(r):
        for k in range(_D // _L):
            buf_a[r, pl.ds(k * _L, _L)] = zf

    rbase = s * _STRIPE
    ncopies = jnp.where(s == _NS - 1, _LAST_COPIES, _STRIPE // _ZB)

    @pl.loop(0, ncopies)
    def _zero_acc(t):
        off = pl.multiple_of(rbase + t * _ZB, 8)
        pltpu.sync_copy(buf_a.at[pl.ds(0, _ZB)], acc.at[pl.ds(off, _ZB)])

    plsc.subcore_barrier()

    def _gate(pk_v):
        for i in range(_CH // _L):
            sl = pl.ds(i * _L, _L)
            r16 = pk_v[0, sl]
            c16 = pk_v[1, sl]
            adj16 = plsc.bitcast(pk_v[2, sl], jnp.float32)
            z = plsc.load_gather(g1_v, [r16]) + plsc.load_gather(g2_v, [c16])
            e2z = jnp.exp(z + z)
            a = 1.0 - 2.0 / (e2z + 1.0)
            val_v[sl] = adj16 * a

    def _scale(buf):
        @pl.loop(0, _CH // _L)
        def _scale_grp(i2):
            v16 = val_v[pl.ds(i2 * _L, _L)]
            for l in range(_L):
                vs = v16[l]
                e = i2 * _L + l
                for k in range(_D // _L):
                    ksl = pl.ds(k * _L, _L)
                    buf[e, ksl] = buf[e, ksl] * vs

    pltpu.sync_copy(pk_hbm.at[wid, 0], pk_a)
    pltpu.async_copy(x_hbm.at[pk_a.at[1]], buf_a, sem_a)

    @pl.loop(0, _NCHUNK // 2)
    def _pair(p):
        j = p * 2
        for b in range(2):
            pk_c, buf_c, sem_c, ssem_c = (pk_a, buf_a, sem_a, ssem_a) \
                if b == 0 else (pk_b, buf_b, sem_b, ssem_b)
            pk_n, buf_n, sem_n, ssem_n = (pk_b, buf_b, sem_b, ssem_b) \
                if b == 0 else (pk_a, buf_a, sem_a, ssem_a)
            nxt = j + b + 1

            @pl.when(nxt < _NCHUNK)
            def _prefetch():
                @pl.when(nxt >= 2)
                def _retire_prev_scatter():
                    pltpu.make_async_copy(
                        buf_n, acc.at[pk_n.at[0]], ssem_n).wait()

                pltpu.sync_copy(pk_hbm.at[wid, nxt], pk_n)
                pltpu.async_copy(x_hbm.at[pk_n.at[1]], buf_n, sem_n)

            _gate(pk_c)
            pltpu.make_async_copy(x_hbm.at[pk_c.at[1]], buf_c, sem_c).wait()
            _scale(buf_c)
            pltpu.async_copy(buf_c, acc.at[pk_c.at[0]], ssem_c, add=True)

    pltpu.make_async_copy(buf_a, acc.at[pk_a.at[0]], ssem_a).wait()
    pltpu.make_async_copy(buf_b, acc.at[pk_b.at[0]], ssem_b).wait()

    plsc.subcore_barrier()

    @pl.loop(0, ncopies)
    def _drain(t):
        off = pl.multiple_of(rbase + t * _ZB, 8)
        pltpu.sync_copy(acc.at[pl.ds(off, _ZB)],
                        out_hbm.at[c, pl.ds(off, _ZB)])


_sc_fala = functools.partial(
    pl.kernel,
    out_type=jax.ShapeDtypeStruct((_NC, _N, _D), jnp.float32),
    mesh=plsc.VectorSubcoreMesh(core_axis_name="c", subcore_axis_name="s",
                                num_cores=_NC, num_subcores=_NS),
    compiler_params=pltpu.CompilerParams(needs_layout_passes=False),
    scratch_types=[
        pltpu.VMEM((3, _CH), jnp.int32),
        pltpu.VMEM((3, _CH), jnp.int32),
        pltpu.VMEM((_N,), jnp.float32),
        pltpu.VMEM((_N,), jnp.float32),
        pltpu.VMEM((_CH,), jnp.float32),
        pltpu.VMEM((_CH, _D), jnp.float32),
        pltpu.VMEM((_CH, _D), jnp.float32),
        pltpu.VMEM_SHARED((_N, _D), jnp.float32),
        pltpu.SemaphoreType.DMA,
        pltpu.SemaphoreType.DMA,
        pltpu.SemaphoreType.DMA,
        pltpu.SemaphoreType.DMA,
    ],
)(_sc_body)


def kernel(x, edge_index, adj_values, gate_w, gate_b):
    w2 = gate_w[:, 0].reshape(2, _D)
    b2 = jnp.concatenate([gate_b, jnp.zeros((1,), jnp.float32)]).reshape(1, 2)
    g = pl.pallas_call(
        _gate_body,
        out_shape=jax.ShapeDtypeStruct((_N, 2), jnp.float32),
    )(x, w2, b2)
    g1 = g[:, 0]
    g2 = g[:, 1]
    packed = jnp.stack(
        [edge_index[0], edge_index[1],
         lax.bitcast_convert_type(adj_values, jnp.int32)], axis=0)
    pk = jnp.pad(packed.reshape(3, _NW, _EPT),
                 ((0, 0), (0, 0), (0, _EPAD - _EPT)))
    pk = pk.reshape(3, _NW, _NCHUNK, _CH).transpose(1, 2, 0, 3)
    partials = _sc_fala(x, g1, g2, pk)
    return pl.pallas_call(
        _sum_body,
        out_shape=jax.ShapeDtypeStruct((_N, _D), jnp.float32),
    )(partials)


# speedup vs baseline: 1.1975x; 1.1975x over previous
import functools

import jax
import jax.numpy as jnp
from jax import lax
from jax.experimental import pallas as pl
from jax.experimental.pallas import tpu as pltpu
from jax.experimental.pallas import tpu_sc as plsc

_N = 10000
_E = 320000
_D = 128
_NC = 2
_NS = 16
_NW = _NC * _NS
_EPT = _E // _NW
_CH = 80
_NCHUNK = 126
_EPAD = _NCHUNK * _CH
_ZB = 80
_STRIPE = 640
_LAST_COPIES = (_N - (_NS - 1) * _STRIPE) // _ZB
_L = 16


def _gate_body(x_ref, w_ref, b_ref, g_ref):
    g_ref[...] = lax.dot_general(
        x_ref[...], w_ref[...], (((1,), (1,)), ((), ())),
        preferred_element_type=jnp.float32) + b_ref[...]


def _sum_body(p_ref, o_ref):
    o_ref[...] = p_ref[0] + p_ref[1]


def _sc_body(x_hbm, g1_hbm, g2_hbm, pk_hbm, out_hbm,
             pk_a, pk_b, g1_v, g2_v, val_v, srow_a, srow_b,
             buf_a, buf_b, acc,
             sem_a, sem_b, ssem_a, ssem_b, psem_a, psem_b):
    c = lax.axis_index("c")
    s = lax.axis_index("s")
    wid = c * _NS + s

    pltpu.sync_copy(g1_hbm, g1_v)
    pltpu.sync_copy(g2_hbm, g2_v)

    zf = jnp.zeros((_L,), jnp.float32)

    @pl.loop(0, _ZB)
    def _zero_buf(r):
        for k in range(_D // _L):
            buf_a[r, pl.ds(k * _L, _L)] = zf

    rbase = s * _STRIPE
    ncopies = jnp.where(s == _NS - 1, _LAST_COPIES, _STRIPE // _ZB)

    @pl.loop(0, ncopies)
    def _zero_acc(t):
        off = pl.multiple_of(rbase + t * _ZB, 8)
        pltpu.sync_copy(buf_a.at[pl.ds(0, _ZB)], acc.at[pl.ds(off, _ZB)])

    plsc.subcore_barrier()

    def _gate(pk_v, srow):
        for i in range(_CH // _L):
            sl = pl.ds(i * _L, _L)
            r16 = pk_v[0, sl]
            c16 = pk_v[1, sl]
            adj16 = plsc.bitcast(pk_v[2, sl], jnp.float32)
            srow[sl] = r16
            z = plsc.load_gather(g1_v, [r16]) + plsc.load_gather(g2_v, [c16])
            e2z = jnp.exp(z + z)
            a = 1.0 - 2.0 / (e2z + 1.0)
            val_v[sl] = adj16 * a

    def _scale(buf):
        @pl.loop(0, _CH // _L)
        def _scale_grp(i2):
            v16 = val_v[pl.ds(i2 * _L, _L)]
            for l in range(_L):
                vs = v16[l]
                e = i2 * _L + l
                for k in range(_D // _L):
                    ksl = pl.ds(k * _L, _L)
                    buf[e, ksl] = buf[e, ksl] * vs

    pltpu.sync_copy(pk_hbm.at[wid, 0], pk_a)
    pltpu.async_copy(x_hbm.at[pk_a.at[1]], buf_a, sem_a)
    pltpu.async_copy(pk_hbm.at[wid, 1], pk_b, psem_b)

    @pl.loop(0, _NCHUNK // 2)
    def _pair(p):
        j = p * 2
        for b in range(2):
            pk_c, buf_c, sem_c, ssem_c, psem_c, srow_c = (
                pk_a, buf_a, sem_a, ssem_a, psem_a, srow_a) if b == 0 else (
                pk_b, buf_b, sem_b, ssem_b, psem_b, srow_b)
            pk_n, buf_n, sem_n, ssem_n, psem_n, srow_n = (
                pk_b, buf_b, sem_b, ssem_b, psem_b, srow_b) if b == 0 else (
                pk_a, buf_a, sem_a, ssem_a, psem_a, srow_a)
            jb = j + b
            nxt = jb + 1

            _gate(pk_c, srow_c)

            @pl.when(nxt < _NCHUNK)
            def _launch_next_gather():
                @pl.when(nxt >= 2)
                def _retire_prev_scatter():
                    pltpu.make_async_copy(
                        buf_n, acc.at[srow_n], ssem_n).wait()

                pltpu.make_async_copy(
                    pk_hbm.at[wid, 0], pk_n, psem_n).wait()
                pltpu.async_copy(x_hbm.at[pk_n.at[1]], buf_n, sem_n)

            pltpu.make_async_copy(x_hbm.at[pk_c.at[1]], buf_c, sem_c).wait()

            @pl.when(jb + 2 < _NCHUNK)
            def _prefetch_pk():
                pltpu.async_copy(pk_hbm.at[wid, jb + 2], pk_c, psem_c)

            _scale(buf_c)
            pltpu.async_copy(buf_c, acc.at[srow_c], ssem_c, add=True)

    pltpu.make_async_copy(buf_a, acc.at[srow_a], ssem_a).wait()
    pltpu.make_async_copy(buf_b, acc.at[srow_b], ssem_b).wait()

    plsc.subcore_barrier()

    @pl.loop(0, ncopies)
    def _drain(t):
        off = pl.multiple_of(rbase + t * _ZB, 8)
        pltpu.sync_copy(acc.at[pl.ds(off, _ZB)],
                        out_hbm.at[c, pl.ds(off, _ZB)])


_sc_fala = functools.partial(
    pl.kernel,
    out_type=jax.ShapeDtypeStruct((_NC, _N, _D), jnp.float32),
    mesh=plsc.VectorSubcoreMesh(core_axis_name="c", subcore_axis_name="s",
                                num_cores=_NC, num_subcores=_NS),
    compiler_params=pltpu.CompilerParams(needs_layout_passes=False),
    scratch_types=[
        pltpu.VMEM((3, _CH), jnp.int32),
        pltpu.VMEM((3, _CH), jnp.int32),
        pltpu.VMEM((_N,), jnp.float32),
        pltpu.VMEM((_N,), jnp.float32),
        pltpu.VMEM((_CH,), jnp.float32),
        pltpu.VMEM((_CH,), jnp.int32),
        pltpu.VMEM((_CH,), jnp.int32),
        pltpu.VMEM((_CH, _D), jnp.float32),
        pltpu.VMEM((_CH, _D), jnp.float32),
        pltpu.VMEM_SHARED((_N, _D), jnp.float32),
        pltpu.SemaphoreType.DMA,
        pltpu.SemaphoreType.DMA,
        pltpu.SemaphoreType.DMA,
        pltpu.SemaphoreType.DMA,
        pltpu.SemaphoreType.DMA,
        pltpu.SemaphoreType.DMA,
    ],
)(_sc_body)


def kernel(x, edge_index, adj_values, gate_w, gate_b):
    w2 = gate_w[:, 0].reshape(2, _D)
    b2 = jnp.concatenate([gate_b, jnp.zeros((1,), jnp.float32)]).reshape(1, 2)
    g = pl.pallas_call(
        _gate_body,
        out_shape=jax.ShapeDtypeStruct((_N, 2), jnp.float32),
    )(x, w2, b2)
    g1 = g[:, 0]
    g2 = g[:, 1]
    packed = jnp.stack(
        [edge_index[0], edge_index[1],
         lax.bitcast_convert_type(adj_values, jnp.int32)], axis=0)
    pk = jnp.pad(packed.reshape(3, _NW, _EPT),
                 ((0, 0), (0, 0), (0, _EPAD - _EPT)))
    pk = pk.reshape(3, _NW, _NCHUNK, _CH).transpose(1, 2, 0, 3)
    partials = _sc_fala(x, g1, g2, pk)
    return pl.pallas_call(
        _sum_body,
        out_shape=jax.ShapeDtypeStruct((_N, _D), jnp.float32),
    )(partials)


# speedup vs baseline: 1.2420x; 1.0371x over previous
import functools

import jax
import jax.numpy as jnp
from jax import lax
from jax.experimental import pallas as pl
from jax.experimental.pallas import tpu as pltpu
from jax.experimental.pallas import tpu_sc as plsc

_N = 10000
_E = 320000
_D = 128
_NC = 2
_NS = 16
_NW = _NC * _NS
_EPT = _E // _NW
_CH = 112
_NCHUNK = 90
_EPAD = _NCHUNK * _CH
_ZB = 80
_STRIPE = 640
_LAST_COPIES = (_N - (_NS - 1) * _STRIPE) // _ZB
_L = 16


def _gate_body(x_ref, w_ref, b_ref, g_ref):
    g_ref[...] = lax.dot_general(
        x_ref[...], w_ref[...], (((1,), (1,)), ((), ())),
        preferred_element_type=jnp.float32) + b_ref[...]


def _sum_body(p_ref, o_ref):
    o_ref[...] = p_ref[0] + p_ref[1]


def _sc_body(x_hbm, g1_hbm, g2_hbm, pk_hbm, out_hbm,
             pk_a, pk_b, g1_v, g2_v, val_v, srow_a, srow_b,
             buf_a, buf_b, acc,
             sem_a, sem_b, ssem_a, ssem_b, psem_a, psem_b):
    c = lax.axis_index("c")
    s = lax.axis_index("s")
    wid = c * _NS + s

    pltpu.sync_copy(g1_hbm, g1_v)
    pltpu.sync_copy(g2_hbm, g2_v)

    zf = jnp.zeros((_L,), jnp.float32)

    @pl.loop(0, _ZB)
    def _zero_buf(r):
        for k in range(_D // _L):
            buf_a[r, pl.ds(k * _L, _L)] = zf

    rbase = s * _STRIPE
    ncopies = jnp.where(s == _NS - 1, _LAST_COPIES, _STRIPE // _ZB)

    @pl.loop(0, ncopies)
    def _zero_acc(t):
        off = pl.multiple_of(rbase + t * _ZB, 8)
        pltpu.sync_copy(buf_a.at[pl.ds(0, _ZB)], acc.at[pl.ds(off, _ZB)])

    plsc.subcore_barrier()

    def _gate(pk_v, srow):
        for i in range(_CH // _L):
            sl = pl.ds(i * _L, _L)
            r16 = pk_v[0, sl]
            c16 = pk_v[1, sl]
            adj16 = plsc.bitcast(pk_v[2, sl], jnp.float32)
            srow[sl] = r16
            z = plsc.load_gather(g1_v, [r16]) + plsc.load_gather(g2_v, [c16])
            e2z = jnp.exp(z + z)
            a = 1.0 - 2.0 / (e2z + 1.0)
            val_v[sl] = adj16 * a

    def _scale(buf):
        @pl.loop(0, _CH // _L)
        def _scale_grp(i2):
            v16 = val_v[pl.ds(i2 * _L, _L)]
            for l in range(_L):
                vs = v16[l]
                e = i2 * _L + l
                for k in range(_D // _L):
                    ksl = pl.ds(k * _L, _L)
                    buf[e, ksl] = buf[e, ksl] * vs

    pltpu.sync_copy(pk_hbm.at[wid, 0], pk_a)
    pltpu.async_copy(x_hbm.at[pk_a.at[1]], buf_a, sem_a)
    pltpu.async_copy(pk_hbm.at[wid, 1], pk_b, psem_b)

    @pl.loop(0, _NCHUNK // 2)
    def _pair(p):
        j = p * 2
        for b in range(2):
            pk_c, buf_c, sem_c, ssem_c, psem_c, srow_c = (
                pk_a, buf_a, sem_a, ssem_a, psem_a, srow_a) if b == 0 else (
                pk_b, buf_b, sem_b, ssem_b, psem_b, srow_b)
            pk_n, buf_n, sem_n, ssem_n, psem_n, srow_n = (
                pk_b, buf_b, sem_b, ssem_b, psem_b, srow_b) if b == 0 else (
                pk_a, buf_a, sem_a, ssem_a, psem_a, srow_a)
            jb = j + b
            nxt = jb + 1

            _gate(pk_c, srow_c)

            @pl.when(nxt < _NCHUNK)
            def _launch_next_gather():
                @pl.when(nxt >= 2)
                def _retire_prev_scatter():
                    pltpu.make_async_copy(
                        buf_n, acc.at[srow_n], ssem_n).wait()

                pltpu.make_async_copy(
                    pk_hbm.at[wid, 0], pk_n, psem_n).wait()
                pltpu.async_copy(x_hbm.at[pk_n.at[1]], buf_n, sem_n)

            pltpu.make_async_copy(x_hbm.at[pk_c.at[1]], buf_c, sem_c).wait()

            @pl.when(jb + 2 < _NCHUNK)
            def _prefetch_pk():
                pltpu.async_copy(pk_hbm.at[wid, jb + 2], pk_c, psem_c)

            _scale(buf_c)
            pltpu.async_copy(buf_c, acc.at[srow_c], ssem_c, add=True)

    pltpu.make_async_copy(buf_a, acc.at[srow_a], ssem_a).wait()
    pltpu.make_async_copy(buf_b, acc.at[srow_b], ssem_b).wait()

    plsc.subcore_barrier()

    @pl.loop(0, ncopies)
    def _drain(t):
        off = pl.multiple_of(rbase + t * _ZB, 8)
        pltpu.sync_copy(acc.at[pl.ds(off, _ZB)],
                        out_hbm.at[c, pl.ds(off, _ZB)])


_sc_fala = functools.partial(
    pl.kernel,
    out_type=jax.ShapeDtypeStruct((_NC, _N, _D), jnp.float32),
    mesh=plsc.VectorSubcoreMesh(core_axis_name="c", subcore_axis_name="s",
                                num_cores=_NC, num_subcores=_NS),
    compiler_params=pltpu.CompilerParams(needs_layout_passes=False),
    scratch_types=[
        pltpu.VMEM((3, _CH), jnp.int32),
        pltpu.VMEM((3, _CH), jnp.int32),
        pltpu.VMEM((_N,), jnp.float32),
        pltpu.VMEM((_N,), jnp.float32),
        pltpu.VMEM((_CH,), jnp.float32),
        pltpu.VMEM((_CH,), jnp.int32),
        pltpu.VMEM((_CH,), jnp.int32),
        pltpu.VMEM((_CH, _D), jnp.float32),
        pltpu.VMEM((_CH, _D), jnp.float32),
        pltpu.VMEM_SHARED((_N, _D), jnp.float32),
        pltpu.SemaphoreType.DMA,
        pltpu.SemaphoreType.DMA,
        pltpu.SemaphoreType.DMA,
        pltpu.SemaphoreType.DMA,
        pltpu.SemaphoreType.DMA,
        pltpu.SemaphoreType.DMA,
    ],
)(_sc_body)


def kernel(x, edge_index, adj_values, gate_w, gate_b):
    w2 = gate_w[:, 0].reshape(2, _D)
    b2 = jnp.concatenate([gate_b, jnp.zeros((1,), jnp.float32)]).reshape(1, 2)
    g = pl.pallas_call(
        _gate_body,
        out_shape=jax.ShapeDtypeStruct((_N, 2), jnp.float32),
    )(x, w2, b2)
    g1 = g[:, 0]
    g2 = g[:, 1]
    packed = jnp.stack(
        [edge_index[0], edge_index[1],
         lax.bitcast_convert_type(adj_values, jnp.int32)], axis=0)
    pk = jnp.pad(packed.reshape(3, _NW, _EPT),
                 ((0, 0), (0, 0), (0, _EPAD - _EPT)))
    pk = pk.reshape(3, _NW, _NCHUNK, _CH).transpose(1, 2, 0, 3)
    partials = _sc_fala(x, g1, g2, pk)
    return pl.pallas_call(
        _sum_body,
        out_shape=jax.ShapeDtypeStruct((_N, _D), jnp.float32),
    )(partials)
